# baseline (speedup 1.0000x reference)
import jax
import jax.numpy as jnp
from jax.experimental import pallas as pl
from jax.experimental.pallas import tpu as pltpu


def _mask_mul_kernel(ch_ref, x_ref, o_ref):
    ch = ch_ref[0]
    c = jax.lax.broadcasted_iota(jnp.int32, x_ref.shape, 0)
    o_ref[...] = jnp.where(c < ch, x_ref[...], 0.0)


def kernel(x, channels):
    B, C, H, W = x.shape
    x2 = x.reshape(B * C, H * W)
    ch = jnp.asarray(channels, jnp.int32).reshape(1)
    out = pl.pallas_call(
        _mask_mul_kernel,
        grid_spec=pltpu.PrefetchScalarGridSpec(
            num_scalar_prefetch=1,
            grid=(B,),
            in_specs=[pl.BlockSpec((C, H * W), lambda b, ch: (b, 0))],
            out_specs=pl.BlockSpec((C, H * W), lambda b, ch: (b, 0)),
        ),
        out_shape=jax.ShapeDtypeStruct((B * C, H * W), x.dtype),
    )(ch, x2)
    return out.reshape(B, C, H, W)
